# baseline (speedup 1.0000x reference)
import functools

import jax
import jax.numpy as jnp
from jax.experimental import pallas as pl
from jax.experimental.pallas import tpu as pltpu

F32 = jnp.float32
BF16 = jnp.bfloat16


def _cross_body(b_ref, u_ref, v_ref, c1_ref, c2_ref, acc2, *, nm):
    m = pl.program_id(0)

    bt = b_ref[...].astype(BF16)
    c1_ref[...] = jnp.dot(bt, u_ref[...],
                          preferred_element_type=F32).astype(c1_ref.dtype)
    contrib = jax.lax.dot_general(
        bt, v_ref[...], (((0,), (0,)), ((), ())), preferred_element_type=F32)

    @pl.when(m == 0)
    def _():
        acc2[...] = contrib

    @pl.when(m > 0)
    def _():
        acc2[...] += contrib

    @pl.when(m == nm - 1)
    def _():
        c2_ref[...] = acc2[...].astype(c2_ref.dtype)


def _cross(b, u, v, *, bm):
    m, k = b.shape
    du, dv = u.shape[1], v.shape[1]
    nm = m // bm
    return pl.pallas_call(
        functools.partial(_cross_body, nm=nm),
        grid=(nm,),
        in_specs=[pl.BlockSpec((bm, k), lambda i: (i, 0)),
                  pl.BlockSpec((k, du), lambda i: (0, 0)),
                  pl.BlockSpec((bm, dv), lambda i: (i, 0))],
        out_specs=[pl.BlockSpec((bm, du), lambda i: (i, 0)),
                   pl.BlockSpec((k, dv), lambda i: (0, 0))],
        out_shape=[jax.ShapeDtypeStruct((m, du), BF16),
                   jax.ShapeDtypeStruct((k, dv), BF16)],
        scratch_shapes=[pltpu.VMEM((k, dv), F32)],
        compiler_params=pltpu.CompilerParams(
            dimension_semantics=("arbitrary",)),
    )(b, u, v)


def _hop_to(out_ref, lbf, h_ref, n, *, mb=1024, cb=512):
    for m0 in range(0, n, mb):
        part = sum(jnp.dot(lbf[m0:m0 + mb, c * cb:(c + 1) * cb],
                           h_ref[c * cb:(c + 1) * cb, :],
                           preferred_element_type=F32)
                   for c in range(n // cb))
        out_ref[m0:m0 + mb, :] = part.astype(BF16)


def _tail_to(y_ref, lbf, r_ref, h1_ref, h2_ref, w_ref, carry_ref, n,
             *, mb=1024, cb=512):
    w = w_ref[...]
    for m0 in range(0, n, mb):
        h3p = sum(jnp.dot(lbf[m0:m0 + mb, c * cb:(c + 1) * cb],
                          h2_ref[c * cb:(c + 1) * cb, :],
                          preferred_element_type=F32)
                  for c in range(n // cb)).astype(BF16)
        y = (jnp.dot(r_ref[m0:m0 + mb, :], w[0:256],
                     preferred_element_type=F32)
             + jnp.dot(h1_ref[m0:m0 + mb, :], w[256:512],
                       preferred_element_type=F32)
             + jnp.dot(h2_ref[m0:m0 + mb, :], w[512:768],
                       preferred_element_type=F32)
             + jnp.dot(h3p, w[768:1024], preferred_element_type=F32))
        if carry_ref is not None:
            y += carry_ref[m0:m0 + mb, :]
        y_ref[m0:m0 + mb, :] = y


def _k1_body(l0_ref, ld_ref, r0_ref, rd_ref, w0_ref, wd_ref,
             y0_ref, pd_ref, lbf0, lbfd, h10, h1b, h2b, *, n0k, ndk, bk):
    s = pl.program_id(0)

    @pl.when(s < n0k)
    def _():
        lt = l0_ref[...].astype(BF16)
        lbf0[pl.ds(s * bk, bk), :] = lt
        h10[pl.ds(s * bk, bk), :] = jnp.dot(
            lt, r0_ref[...], preferred_element_type=F32).astype(BF16)

    @pl.when((s >= n0k) & (s < n0k + ndk))
    def _():
        k = s - n0k
        lt = ld_ref[...].astype(BF16)
        lbfd[pl.ds(k * bk, bk), :] = lt
        h1b[pl.ds(k * bk, bk), :] = jnp.dot(
            lt, rd_ref[...], preferred_element_type=F32).astype(BF16)

    @pl.when(s == n0k)
    def _():
        _hop_to(h2b, lbf0, h10, 1024)
        _tail_to(y0_ref, lbf0, r0_ref, h10, h2b, w0_ref, None, 1024)

    @pl.when(s == n0k + ndk - 1)
    def _():
        _hop_to(h2b, lbfd, h1b, 3072)
        _tail_to(pd_ref, lbfd, rd_ref, h1b, h2b, wd_ref, None, 3072)


def _k1(l0, ld, r0, rd, w0, wd, *, bk=512):
    n0, nd = l0.shape[0], ld.shape[0]
    n0k, ndk = n0 // bk, nd // bk
    c0 = n0k - 1
    cd = ndk - 1
    return pl.pallas_call(
        functools.partial(_k1_body, n0k=n0k, ndk=ndk, bk=bk),
        grid=(n0k + ndk,),
        in_specs=[
            pl.BlockSpec((bk, n0), lambda s: (jnp.clip(s, 0, c0), 0)),
            pl.BlockSpec((bk, nd), lambda s: (jnp.clip(s - c0 - 1, 0, cd), 0)),
            pl.BlockSpec((n0, 256), lambda s: (0, 0)),
            pl.BlockSpec((nd, 256), lambda s: (0, 0)),
            pl.BlockSpec((1024, 128), lambda s: (0, 0)),
            pl.BlockSpec((1024, 128), lambda s: (0, 0)),
        ],
        out_specs=[pl.BlockSpec((n0, 128), lambda s: (0, 0)),
                   pl.BlockSpec((nd, 128), lambda s: (0, 0))],
        out_shape=[jax.ShapeDtypeStruct((n0, 128), F32),
                   jax.ShapeDtypeStruct((nd, 128), F32)],
        scratch_shapes=[pltpu.VMEM((n0, n0), BF16),
                        pltpu.VMEM((nd, nd), BF16),
                        pltpu.VMEM((n0, 256), BF16),
                        pltpu.VMEM((nd, 256), BF16),
                        pltpu.VMEM((nd, 256), BF16)],
        compiler_params=pltpu.CompilerParams(
            dimension_semantics=("arbitrary",)),
    )(l0, ld, r0, rd, w0, wd)


def _k2_body(lu_ref, l2_ref, ru_ref, r2_ref, wu_ref, w2_ref, pd_ref,
             y1_ref, y2_ref, lbfu, lbf2, h1u, h2u, h12,
             *, nuk, n2k, bku, bk2):
    s = pl.program_id(0)
    last = nuk + n2k

    @pl.when(s < nuk)
    def _():
        lt = lu_ref[...].astype(BF16)
        lbfu[pl.ds(s * bku, bku), :] = lt
        h1u[pl.ds(s * bku, bku), :] = jnp.dot(
            lt, ru_ref[...], preferred_element_type=F32).astype(BF16)

    @pl.when((s >= nuk) & (s < last))
    def _():
        k = s - nuk
        lt = l2_ref[...].astype(BF16)
        lbf2[pl.ds(k * bk2, bk2), :] = lt
        h12[pl.ds(k * bk2, bk2), :] = jnp.dot(
            lt, r2_ref[...], preferred_element_type=F32).astype(BF16)

    for c in range(6):
        @pl.when(s == nuk + c)
        def _(c=c):
            m0 = c * 512
            part = sum(jnp.dot(lbfu[m0:m0 + 512, j * 512:(j + 1) * 512],
                               h1u[j * 512:(j + 1) * 512, :],
                               preferred_element_type=F32)
                       for j in range(6))
            h2u[m0:m0 + 512, :] = part.astype(BF16)

    for c in range(6):
        @pl.when(s == nuk + 6 + c)
        def _(c=c):
            m0 = c * 512
            wu = wu_ref[...]
            h3p = sum(jnp.dot(lbfu[m0:m0 + 512, j * 512:(j + 1) * 512],
                              h2u[j * 512:(j + 1) * 512, :],
                              preferred_element_type=F32)
                      for j in range(6)).astype(BF16)
            y1_ref[m0:m0 + 512, :] = (
                jnp.dot(ru_ref[m0:m0 + 512, :], wu[0:256],
                        preferred_element_type=F32)
                + jnp.dot(h1u[m0:m0 + 512, :], wu[256:512],
                          preferred_element_type=F32)
                + jnp.dot(h2u[m0:m0 + 512, :], wu[512:768],
                          preferred_element_type=F32)
                + jnp.dot(h3p, wu[768:1024], preferred_element_type=F32)
                + pd_ref[m0:m0 + 512, :])

    @pl.when(s == last)
    def _():
        _hop_to(h2u, lbf2, h12, 2048)
        _tail_to(y2_ref, lbf2, r2_ref, h12, h2u, w2_ref, None, 2048)


def _k2(lu, l2, ru, r2, wu, w2, pd, *, bku=512, bk2=128):
    nu, n2 = lu.shape[0], l2.shape[0]
    nuk, n2k = nu // bku, n2 // bk2
    cu = nuk - 1
    c2 = n2k - 1
    return pl.pallas_call(
        functools.partial(_k2_body, nuk=nuk, n2k=n2k, bku=bku, bk2=bk2),
        grid=(nuk + n2k + 1,),
        in_specs=[
            pl.BlockSpec((bku, nu), lambda s: (jnp.clip(s, 0, cu), 0)),
            pl.BlockSpec((bk2, n2), lambda s: (jnp.clip(s - cu - 1, 0, c2), 0)),
            pl.BlockSpec((nu, 256), lambda s: (0, 0)),
            pl.BlockSpec((n2, 256), lambda s: (0, 0)),
            pl.BlockSpec((1024, 128), lambda s: (0, 0)),
            pl.BlockSpec((1024, 128), lambda s: (0, 0)),
            pl.BlockSpec((nu, 128), lambda s: (0, 0)),
        ],
        out_specs=[pl.BlockSpec((nu, 128), lambda s: (0, 0)),
                   pl.BlockSpec((n2, 128), lambda s: (0, 0))],
        out_shape=[jax.ShapeDtypeStruct((nu, 128), F32),
                   jax.ShapeDtypeStruct((n2, 128), F32)],
        scratch_shapes=[pltpu.VMEM((nu, nu), BF16),
                        pltpu.VMEM((n2, n2), BF16),
                        pltpu.VMEM((nu, 256), BF16),
                        pltpu.VMEM((nu, 256), BF16),
                        pltpu.VMEM((n2, 256), BF16)],
        compiler_params=pltpu.CompilerParams(
            dimension_semantics=("arbitrary",)),
    )(lu, l2, ru, r2, wu, w2, pd)


def _wstack(w, pairs):
    zero = jnp.zeros(w.shape[:2], w.dtype)
    blocks = []
    for a, b in pairs:
        blocks.append(zero if a is None else w[:, :, a])
        blocks.append(zero if b is None else w[:, :, b])
    return jnp.concatenate(blocks, axis=0).astype(BF16)


def kernel(x_0, x_1, x_2, laplacian_0, laplacian_down_1, laplacian_up_1,
           laplacian_2, b1, b2, weight_0, weight_1, weight_2):
    x0 = x_0.astype(BF16)
    x1 = x_1.astype(BF16)
    x2 = x_2.astype(BF16)

    t10, t01 = _cross(b1, x1, x0, bm=256)
    t21, t12 = _cross(b2, x2, x1, bm=512)

    r0 = jnp.concatenate([x0, t10], axis=1)
    rd = jnp.concatenate([t01, x1], axis=1)
    ru = jnp.concatenate([x1, t21], axis=1)
    r2 = jnp.concatenate([x2, t12], axis=1)

    W0 = _wstack(weight_0, ((0, 4), (1, 5), (2, 6), (3, 7)))
    Wd = _wstack(weight_1, ((0, 4), (1, 5), (2, 6), (3, 7)))
    Wu = _wstack(weight_1, ((None, 11), (8, 12), (9, 13), (10, 14)))
    W2 = _wstack(weight_2, ((0, 4), (1, 5), (2, 6), (3, 7)))

    y_0, p_d = _k1(laplacian_0, laplacian_down_1, r0, rd, W0, Wd)
    y_1, y_2 = _k2(laplacian_up_1, laplacian_2, ru, r2, Wu, W2, p_d)

    return y_0, y_1, y_2


# speedup vs baseline: 1.0098x; 1.0098x over previous
import functools

import jax
import jax.numpy as jnp
from jax.experimental import pallas as pl
from jax.experimental.pallas import tpu as pltpu

F32 = jnp.float32
BF16 = jnp.bfloat16


def _crosses_body(b1_ref, b2_ref, x0_ref, x1f_ref, x1v_ref, x2_ref,
                  t10_ref, t01_ref, t21_ref, t12_ref, acc1, acc2,
                  *, nm1, nm2):
    s = pl.program_id(0)

    @pl.when(s < nm1)
    def _():
        bt = b1_ref[...].astype(BF16)
        t10_ref[...] = jnp.dot(bt, x1f_ref[...],
                               preferred_element_type=F32).astype(BF16)
        contrib = jax.lax.dot_general(
            bt, x0_ref[...], (((0,), (0,)), ((), ())),
            preferred_element_type=F32)

        @pl.when(s == 0)
        def _():
            acc1[...] = contrib

        @pl.when(s > 0)
        def _():
            acc1[...] += contrib

        @pl.when(s == nm1 - 1)
        def _():
            t01_ref[...] = acc1[...].astype(BF16)

    @pl.when(s >= nm1)
    def _():
        m = s - nm1
        bt = b2_ref[...].astype(BF16)
        t21_ref[...] = jnp.dot(bt, x2_ref[...],
                               preferred_element_type=F32).astype(BF16)
        contrib = jax.lax.dot_general(
            bt, x1v_ref[...], (((0,), (0,)), ((), ())),
            preferred_element_type=F32)

        @pl.when(m == 0)
        def _():
            acc2[...] = contrib

        @pl.when(m > 0)
        def _():
            acc2[...] += contrib

        @pl.when(m == nm2 - 1)
        def _():
            t12_ref[...] = acc2[...].astype(BF16)


def _crosses(b1, b2, x0, x1, x2, *, bm1=512, bm2=1024):
    m1, k1 = b1.shape
    m2, k2 = b2.shape
    nm1, nm2 = m1 // bm1, m2 // bm2
    c1m = nm1 - 1
    c2m = nm2 - 1
    return pl.pallas_call(
        functools.partial(_crosses_body, nm1=nm1, nm2=nm2),
        grid=(nm1 + nm2,),
        in_specs=[
            pl.BlockSpec((bm1, k1), lambda s: (jnp.clip(s, 0, c1m), 0)),
            pl.BlockSpec((bm2, k2),
                         lambda s: (jnp.clip(s - c1m - 1, 0, c2m), 0)),
            pl.BlockSpec((bm1, 128), lambda s: (jnp.clip(s, 0, c1m), 0)),
            pl.BlockSpec((k1, 128), lambda s: (0, 0)),
            pl.BlockSpec((bm2, 128),
                         lambda s: (jnp.clip(s - c1m - 1, 0, c2m), 0)),
            pl.BlockSpec((k2, 128), lambda s: (0, 0)),
        ],
        out_specs=[pl.BlockSpec((bm1, 128), lambda s: (jnp.clip(s, 0, c1m), 0)),
                   pl.BlockSpec((k1, 128), lambda s: (0, 0)),
                   pl.BlockSpec((bm2, 128),
                                lambda s: (jnp.clip(s - c1m - 1, 0, c2m), 0)),
                   pl.BlockSpec((k2, 128), lambda s: (0, 0))],
        out_shape=[jax.ShapeDtypeStruct((m1, 128), BF16),
                   jax.ShapeDtypeStruct((k1, 128), BF16),
                   jax.ShapeDtypeStruct((m2, 128), BF16),
                   jax.ShapeDtypeStruct((k2, 128), BF16)],
        scratch_shapes=[pltpu.VMEM((k1, 128), F32),
                        pltpu.VMEM((k2, 128), F32)],
        compiler_params=pltpu.CompilerParams(
            dimension_semantics=("arbitrary",)),
    )(b1, b2, x0, x1, x1, x2)


def _hop_to(out_ref, lbf, h_ref, n, *, mb=1024, cb=512):
    for m0 in range(0, n, mb):
        part = sum(jnp.dot(lbf[m0:m0 + mb, c * cb:(c + 1) * cb],
                           h_ref[c * cb:(c + 1) * cb, :],
                           preferred_element_type=F32)
                   for c in range(n // cb))
        out_ref[m0:m0 + mb, :] = part.astype(BF16)


def _tail_to(y_ref, lbf, r_ref, h1_ref, h2_ref, w_ref, carry_ref, n,
             *, mb=1024, cb=512):
    w = w_ref[...]
    for m0 in range(0, n, mb):
        h3p = sum(jnp.dot(lbf[m0:m0 + mb, c * cb:(c + 1) * cb],
                          h2_ref[c * cb:(c + 1) * cb, :],
                          preferred_element_type=F32)
                  for c in range(n // cb)).astype(BF16)
        y = (jnp.dot(r_ref[m0:m0 + mb, :], w[0:256],
                     preferred_element_type=F32)
             + jnp.dot(h1_ref[m0:m0 + mb, :], w[256:512],
                       preferred_element_type=F32)
             + jnp.dot(h2_ref[m0:m0 + mb, :], w[512:768],
                       preferred_element_type=F32)
             + jnp.dot(h3p, w[768:1024], preferred_element_type=F32))
        if carry_ref is not None:
            y += carry_ref[m0:m0 + mb, :]
        y_ref[m0:m0 + mb, :] = y


def _k1_body(l0_ref, ld_ref, r0_ref, rd_ref, w0_ref, wd_ref,
             y0_ref, pd_ref, lbf0, lbfd, h10, h1b, h2b, *, n0k, ndk, bk):
    s = pl.program_id(0)

    @pl.when(s < n0k)
    def _():
        lt = l0_ref[...].astype(BF16)
        lbf0[pl.ds(s * bk, bk), :] = lt
        h10[pl.ds(s * bk, bk), :] = jnp.dot(
            lt, r0_ref[...], preferred_element_type=F32).astype(BF16)

    @pl.when((s >= n0k) & (s < n0k + ndk))
    def _():
        k = s - n0k
        lt = ld_ref[...].astype(BF16)
        lbfd[pl.ds(k * bk, bk), :] = lt
        h1b[pl.ds(k * bk, bk), :] = jnp.dot(
            lt, rd_ref[...], preferred_element_type=F32).astype(BF16)

    @pl.when(s == n0k)
    def _():
        _hop_to(h2b, lbf0, h10, 1024)
        _tail_to(y0_ref, lbf0, r0_ref, h10, h2b, w0_ref, None, 1024)

    @pl.when(s == n0k + ndk - 1)
    def _():
        _hop_to(h2b, lbfd, h1b, 3072)
        _tail_to(pd_ref, lbfd, rd_ref, h1b, h2b, wd_ref, None, 3072)


def _k1(l0, ld, r0, rd, w0, wd, *, bk=512):
    n0, nd = l0.shape[0], ld.shape[0]
    n0k, ndk = n0 // bk, nd // bk
    c0 = n0k - 1
    cd = ndk - 1
    return pl.pallas_call(
        functools.partial(_k1_body, n0k=n0k, ndk=ndk, bk=bk),
        grid=(n0k + ndk,),
        in_specs=[
            pl.BlockSpec((bk, n0), lambda s: (jnp.clip(s, 0, c0), 0)),
            pl.BlockSpec((bk, nd), lambda s: (jnp.clip(s - c0 - 1, 0, cd), 0)),
            pl.BlockSpec((n0, 256), lambda s: (0, 0)),
            pl.BlockSpec((nd, 256), lambda s: (0, 0)),
            pl.BlockSpec((1024, 128), lambda s: (0, 0)),
            pl.BlockSpec((1024, 128), lambda s: (0, 0)),
        ],
        out_specs=[pl.BlockSpec((n0, 128), lambda s: (0, 0)),
                   pl.BlockSpec((nd, 128), lambda s: (0, 0))],
        out_shape=[jax.ShapeDtypeStruct((n0, 128), F32),
                   jax.ShapeDtypeStruct((nd, 128), F32)],
        scratch_shapes=[pltpu.VMEM((n0, n0), BF16),
                        pltpu.VMEM((nd, nd), BF16),
                        pltpu.VMEM((n0, 256), BF16),
                        pltpu.VMEM((nd, 256), BF16),
                        pltpu.VMEM((nd, 256), BF16)],
        compiler_params=pltpu.CompilerParams(
            dimension_semantics=("arbitrary",)),
    )(l0, ld, r0, rd, w0, wd)


def _k2_body(lu_ref, l2_ref, ru_ref, r2_ref, wu_ref, w2_ref, pd_ref,
             y1_ref, y2_ref, lbfu, lbf2, h1u, h2u, h12,
             *, nuk, n2k, bku, bk2):
    s = pl.program_id(0)
    last = nuk + n2k

    @pl.when(s < nuk)
    def _():
        lt = lu_ref[...].astype(BF16)
        lbfu[pl.ds(s * bku, bku), :] = lt
        h1u[pl.ds(s * bku, bku), :] = jnp.dot(
            lt, ru_ref[...], preferred_element_type=F32).astype(BF16)

    @pl.when((s >= nuk) & (s < last))
    def _():
        k = s - nuk
        lt = l2_ref[...].astype(BF16)
        lbf2[pl.ds(k * bk2, bk2), :] = lt
        h12[pl.ds(k * bk2, bk2), :] = jnp.dot(
            lt, r2_ref[...], preferred_element_type=F32).astype(BF16)

    for c in range(6):
        @pl.when(s == nuk + c)
        def _(c=c):
            m0 = c * 512
            part = sum(jnp.dot(lbfu[m0:m0 + 512, j * 512:(j + 1) * 512],
                               h1u[j * 512:(j + 1) * 512, :],
                               preferred_element_type=F32)
                       for j in range(6))
            h2u[m0:m0 + 512, :] = part.astype(BF16)

    for c in range(6):
        @pl.when(s == nuk + 6 + c)
        def _(c=c):
            m0 = c * 512
            wu = wu_ref[...]
            h3p = sum(jnp.dot(lbfu[m0:m0 + 512, j * 512:(j + 1) * 512],
                              h2u[j * 512:(j + 1) * 512, :],
                              preferred_element_type=F32)
                      for j in range(6)).astype(BF16)
            y1_ref[m0:m0 + 512, :] = (
                jnp.dot(ru_ref[m0:m0 + 512, :], wu[0:256],
                        preferred_element_type=F32)
                + jnp.dot(h1u[m0:m0 + 512, :], wu[256:512],
                          preferred_element_type=F32)
                + jnp.dot(h2u[m0:m0 + 512, :], wu[512:768],
                          preferred_element_type=F32)
                + jnp.dot(h3p, wu[768:1024], preferred_element_type=F32)
                + pd_ref[m0:m0 + 512, :])

    @pl.when(s == last)
    def _():
        _hop_to(h2u, lbf2, h12, 2048)
        _tail_to(y2_ref, lbf2, r2_ref, h12, h2u, w2_ref, None, 2048)


def _k2(lu, l2, ru, r2, wu, w2, pd, *, bku=512, bk2=128):
    nu, n2 = lu.shape[0], l2.shape[0]
    nuk, n2k = nu // bku, n2 // bk2
    cu = nuk - 1
    c2 = n2k - 1
    return pl.pallas_call(
        functools.partial(_k2_body, nuk=nuk, n2k=n2k, bku=bku, bk2=bk2),
        grid=(nuk + n2k + 1,),
        in_specs=[
            pl.BlockSpec((bku, nu), lambda s: (jnp.clip(s, 0, cu), 0)),
            pl.BlockSpec((bk2, n2), lambda s: (jnp.clip(s - cu - 1, 0, c2), 0)),
            pl.BlockSpec((nu, 256), lambda s: (0, 0)),
            pl.BlockSpec((n2, 256), lambda s: (0, 0)),
            pl.BlockSpec((1024, 128), lambda s: (0, 0)),
            pl.BlockSpec((1024, 128), lambda s: (0, 0)),
            pl.BlockSpec((nu, 128), lambda s: (0, 0)),
        ],
        out_specs=[pl.BlockSpec((nu, 128), lambda s: (0, 0)),
                   pl.BlockSpec((n2, 128), lambda s: (0, 0))],
        out_shape=[jax.ShapeDtypeStruct((nu, 128), F32),
                   jax.ShapeDtypeStruct((n2, 128), F32)],
        scratch_shapes=[pltpu.VMEM((nu, nu), BF16),
                        pltpu.VMEM((n2, n2), BF16),
                        pltpu.VMEM((nu, 256), BF16),
                        pltpu.VMEM((nu, 256), BF16),
                        pltpu.VMEM((n2, 256), BF16)],
        compiler_params=pltpu.CompilerParams(
            dimension_semantics=("arbitrary",)),
    )(lu, l2, ru, r2, wu, w2, pd)


def _wstack(w, pairs):
    zero = jnp.zeros(w.shape[:2], w.dtype)
    blocks = []
    for a, b in pairs:
        blocks.append(zero if a is None else w[:, :, a])
        blocks.append(zero if b is None else w[:, :, b])
    return jnp.concatenate(blocks, axis=0).astype(BF16)


def kernel(x_0, x_1, x_2, laplacian_0, laplacian_down_1, laplacian_up_1,
           laplacian_2, b1, b2, weight_0, weight_1, weight_2):
    x0 = x_0.astype(BF16)
    x1 = x_1.astype(BF16)
    x2 = x_2.astype(BF16)

    t10, t01, t21, t12 = _crosses(b1, b2, x0, x1, x2)

    r0 = jnp.concatenate([x0, t10], axis=1)
    rd = jnp.concatenate([t01, x1], axis=1)
    ru = jnp.concatenate([x1, t21], axis=1)
    r2 = jnp.concatenate([x2, t12], axis=1)

    W0 = _wstack(weight_0, ((0, 4), (1, 5), (2, 6), (3, 7)))
    Wd = _wstack(weight_1, ((0, 4), (1, 5), (2, 6), (3, 7)))
    Wu = _wstack(weight_1, ((None, 11), (8, 12), (9, 13), (10, 14)))
    W2 = _wstack(weight_2, ((0, 4), (1, 5), (2, 6), (3, 7)))

    y_0, p_d = _k1(laplacian_0, laplacian_down_1, r0, rd, W0, Wd)
    y_1, y_2 = _k2(laplacian_up_1, laplacian_2, ru, r2, Wu, W2, p_d)

    return y_0, y_1, y_2
